# baseline (speedup 1.0000x reference)
import functools

import jax
import jax.numpy as jnp
from jax import lax
from jax.experimental import pallas as pl
from jax.experimental.pallas import tpu as pltpu
from jax.experimental.pallas import tpu_sc as plsc

MODEL_DIM = 64
SEQ = 200
LANES = 16
CHUNK = 128


@functools.cache
def _build(n_tokens, seq, vocab, model_dim):
    info = plsc.get_sparse_core_info()
    nw = info.num_cores * info.num_subcores
    assert n_tokens % (nw * CHUNK) == 0
    per_w = n_tokens // nw
    n_chunks = per_w // CHUNK
    n_vecs = model_dim // LANES

    mesh = plsc.VectorSubcoreMesh(core_axis_name="c", subcore_axis_name="s")

    @functools.partial(
        pl.kernel,
        out_type=jax.ShapeDtypeStruct((n_tokens, model_dim), jnp.float32),
        mesh=mesh,
        scratch_types=[
            pltpu.VMEM((seq, model_dim), jnp.float32),
            pltpu.VMEM((CHUNK,), jnp.int32),
            pltpu.VMEM((CHUNK, model_dim), jnp.float32),
            pltpu.SemaphoreType.DMA,
        ],
        compiler_params=pltpu.CompilerParams(use_tc_tiling_on_sc=False),
    )
    def embed(idx_hbm, table_hbm, pos_hbm, out_hbm, pos_v, idx_v, rows_v, sem):
        wid = lax.axis_index("s") * info.num_cores + lax.axis_index("c")
        base = wid * per_w
        pltpu.sync_copy(pos_hbm, pos_v)

        def chunk_body(c, _):
            off = base + c * CHUNK
            pltpu.sync_copy(idx_hbm.at[pl.ds(off, CHUNK)], idx_v)
            pltpu.async_copy(table_hbm.at[idx_v], rows_v, sem).wait()
            pbase = lax.rem(c * CHUNK, seq)

            def row_body(r, _):
                pr = lax.rem(pbase + r, seq)
                for j in range(n_vecs):
                    sl = pl.ds(j * LANES, LANES)
                    rows_v[r, sl] = rows_v[r, sl] * 8.0 + pos_v[pr, sl]
                return 0

            lax.fori_loop(0, CHUNK, row_body, 0)
            pltpu.sync_copy(rows_v, out_hbm.at[pl.ds(off, CHUNK)])
            return 0

        lax.fori_loop(0, n_chunks, chunk_body, 0)

    return embed


def kernel(inp, table, pos_encoding):
    batch, seq = inp.shape
    vocab, model_dim = table.shape
    idx_flat = inp.reshape(-1)
    pos2d = pos_encoding[0, :seq, :]
    embed = _build(batch * seq, seq, vocab, model_dim)
    out2d = embed(idx_flat, table, pos2d)
    return out2d.reshape(batch, seq, model_dim)


# speedup vs baseline: 1.0777x; 1.0777x over previous
import functools

import jax
import jax.numpy as jnp
from jax import lax
from jax.experimental import pallas as pl
from jax.experimental.pallas import tpu as pltpu
from jax.experimental.pallas import tpu_sc as plsc

LANES = 16
CHUNK = 128
NBUF = 5


@functools.cache
def _build(n_tokens, seq, vocab, model_dim):
    info = plsc.get_sparse_core_info()
    nw = info.num_cores * info.num_subcores
    assert n_tokens % (nw * CHUNK * NBUF) == 0
    per_w = n_tokens // nw
    n_chunks = per_w // CHUNK
    n_outer = n_chunks // NBUF
    n_vecs = model_dim // LANES
    pos_rows = seq + CHUNK

    mesh = plsc.VectorSubcoreMesh(core_axis_name="c", subcore_axis_name="s")

    @functools.partial(
        pl.kernel,
        out_type=jax.ShapeDtypeStruct((n_tokens, model_dim), jnp.float32),
        mesh=mesh,
        scratch_types=[
            pltpu.VMEM((pos_rows, model_dim), jnp.float32),
            pltpu.VMEM((n_chunks, CHUNK), jnp.int32),
            pltpu.VMEM((NBUF, CHUNK, model_dim), jnp.float32),
            pltpu.VMEM((NBUF, CHUNK, model_dim), jnp.float32),
            pltpu.SemaphoreType.DMA((NBUF,)),
            pltpu.SemaphoreType.DMA((NBUF,)),
        ],
        compiler_params=pltpu.CompilerParams(use_tc_tiling_on_sc=False),
    )
    def embed(idx_hbm, table_hbm, pos_hbm, out_hbm,
              pos_v, idx_v, rows_v, out_v, gsem, osem):
        wid = lax.axis_index("s") * info.num_cores + lax.axis_index("c")
        base = wid * per_w
        pltpu.sync_copy(pos_hbm, pos_v)
        pltpu.sync_copy(idx_hbm.at[pl.ds(wid * n_chunks, n_chunks)], idx_v)

        def fire_gather(c, b):
            pltpu.async_copy(table_hbm.at[idx_v.at[c]], rows_v.at[b],
                             gsem.at[b])

        for b in range(NBUF):
            fire_gather(b, b)

        def outer(cc, _):
            for b in range(NBUF):
                c = cc * NBUF + b
                pltpu.make_async_copy(
                    table_hbm.at[idx_v.at[c]], rows_v.at[b], gsem.at[b]
                ).wait()

                @pl.when(cc > 0)
                def _():
                    pltpu.make_async_copy(
                        out_v.at[b], out_hbm.at[pl.ds(base, CHUNK)],
                        osem.at[b]).wait()

                pbase = lax.rem(c * CHUNK, seq)

                def row_body(r, _):
                    p = pbase + r
                    for j in range(n_vecs):
                        sl = pl.ds(j * LANES, LANES)
                        out_v[b, r, sl] = rows_v[b, r, sl] * 8.0 + pos_v[p, sl]
                    return 0

                lax.fori_loop(0, CHUNK, row_body, 0)

                @pl.when(cc < n_outer - 1)
                def _():
                    fire_gather(c + NBUF, b)

                pltpu.async_copy(out_v.at[b],
                                 out_hbm.at[pl.ds(base + c * CHUNK, CHUNK)],
                                 osem.at[b])
            return 0

        lax.fori_loop(0, n_outer, outer, 0)
        for b in range(NBUF):
            pltpu.make_async_copy(
                out_v.at[b], out_hbm.at[pl.ds(base, CHUNK)], osem.at[b]
            ).wait()

    return embed


def kernel(inp, table, pos_encoding):
    batch, seq = inp.shape
    vocab, model_dim = table.shape
    idx2d = inp.reshape(-1, CHUNK)
    pos2d = pos_encoding[0, :seq, :]
    pos_ext = jnp.concatenate([pos2d, pos2d[:CHUNK]], axis=0)
    embed = _build(batch * seq, seq, vocab, model_dim)
    out2d = embed(idx2d, table, pos_ext)
    return out2d.reshape(batch, seq, model_dim)


# speedup vs baseline: 1.2184x; 1.1306x over previous
import functools

import jax
import jax.numpy as jnp
from jax import lax
from jax.experimental import pallas as pl
from jax.experimental.pallas import tpu as pltpu
from jax.experimental.pallas import tpu_sc as plsc

LANES = 16
GCHUNK = 128
NBUF = 4


@functools.cache
def _build(batch, seq, vocab, model_dim):
    info = plsc.get_sparse_core_info()
    nw = info.num_cores * info.num_subcores
    assert batch % (nw * NBUF) == 0
    seq_per_w = batch // nw
    n_outer = seq_per_w // NBUF
    n_vecs = model_dim // LANES
    tail = seq - GCHUNK

    mesh = plsc.VectorSubcoreMesh(core_axis_name="c", subcore_axis_name="s")

    @functools.partial(
        pl.kernel,
        out_type=jax.ShapeDtypeStruct((batch, seq, model_dim), jnp.float32),
        mesh=mesh,
        scratch_types=[
            pltpu.VMEM((seq, model_dim), jnp.float32),
            pltpu.VMEM((seq_per_w, seq), jnp.int32),
            pltpu.VMEM((NBUF, seq, model_dim), jnp.float32),
            pltpu.VMEM((NBUF, seq, model_dim), jnp.float32),
            pltpu.SemaphoreType.DMA((NBUF,)),
            pltpu.SemaphoreType.DMA((NBUF,)),
        ],
        compiler_params=pltpu.CompilerParams(use_tc_tiling_on_sc=False),
    )
    def embed(idx_hbm, table_hbm, pos_hbm, out_hbm,
              pos_v, idx_v, rows_v, out_v, gsem, osem):
        wid = lax.axis_index("s") * info.num_cores + lax.axis_index("c")
        base = wid * seq_per_w
        pltpu.sync_copy(pos_hbm, pos_v)
        pltpu.sync_copy(idx_hbm.at[pl.ds(base, seq_per_w)], idx_v)

        def fire_gather(s, b):
            pltpu.async_copy(table_hbm.at[idx_v.at[s, pl.ds(0, GCHUNK)]],
                             rows_v.at[b, pl.ds(0, GCHUNK)], gsem.at[b])
            pltpu.async_copy(table_hbm.at[idx_v.at[s, pl.ds(GCHUNK, tail)]],
                             rows_v.at[b, pl.ds(GCHUNK, tail)], gsem.at[b])

        def wait_gather(s, b):
            pltpu.make_async_copy(
                table_hbm.at[idx_v.at[s, pl.ds(0, GCHUNK)]],
                rows_v.at[b, pl.ds(0, GCHUNK)], gsem.at[b]).wait()
            pltpu.make_async_copy(
                table_hbm.at[idx_v.at[s, pl.ds(GCHUNK, tail)]],
                rows_v.at[b, pl.ds(GCHUNK, tail)], gsem.at[b]).wait()

        for b in range(NBUF):
            fire_gather(b, b)

        def outer(cc, _):
            for b in range(NBUF):
                s = cc * NBUF + b
                wait_gather(s, b)

                @pl.when(cc > 0)
                def _():
                    pltpu.make_async_copy(
                        out_v.at[b], out_hbm.at[base], osem.at[b]).wait()

                def row_body(r, _):
                    for j in range(n_vecs):
                        sl = pl.ds(j * LANES, LANES)
                        out_v[b, r, sl] = rows_v[b, r, sl] * 8.0 + pos_v[r, sl]
                    return 0

                lax.fori_loop(0, seq, row_body, 0)

                @pl.when(cc < n_outer - 1)
                def _():
                    fire_gather(s + NBUF, b)

                pltpu.async_copy(out_v.at[b], out_hbm.at[base + s],
                                 osem.at[b])
            return 0

        lax.fori_loop(0, n_outer, outer, 0)
        for b in range(NBUF):
            pltpu.make_async_copy(
                out_v.at[b], out_hbm.at[base], osem.at[b]).wait()

    return embed


def kernel(inp, table, pos_encoding):
    batch, seq = inp.shape
    vocab, model_dim = table.shape
    pos2d = pos_encoding[0, :seq, :]
    embed = _build(batch, seq, vocab, model_dim)
    return embed(inp, table, pos2d)
